# baseline (speedup 1.0000x reference)
import functools

import jax
import jax.numpy as jnp
from jax import lax
from jax.experimental import pallas as pl
from jax.experimental.pallas import tpu as pltpu
from jax.experimental.pallas import tpu_sc as plsc

HIDDEN = 128
N_NODES = 10000
N_EDGES = 320000

NC = 2
NS = 16
NW = NC * NS
EW = N_EDGES // NW
CH = 80
NCH = EW // CH
LANES = 16
NGRP = HIDDEN // LANES


def _gates_tc_kernel(h_ref, w_ref, b_ref, g_ref):
    hb = h_ref[...]
    u = lax.dot_general(hb, w_ref[...], (((1,), (1,)), ((), ())),
                        preferred_element_type=jnp.float32) + b_ref[...]
    g_ref[:, :HIDDEN] = hb
    g_ref[:, HIDDEN:] = u


def _build_gates(h, W_r, b2):
    rows = 1000
    return pl.pallas_call(
        _gates_tc_kernel,
        grid=(N_NODES // rows,),
        in_specs=[
            pl.BlockSpec((rows, HIDDEN), lambda i: (i, 0)),
            pl.BlockSpec((HIDDEN, HIDDEN), lambda i: (0, 0)),
            pl.BlockSpec((1, HIDDEN), lambda i: (0, 0)),
        ],
        out_specs=pl.BlockSpec((rows, 2 * HIDDEN), lambda i: (i, 0)),
        out_shape=jax.ShapeDtypeStruct((N_NODES, 2 * HIDDEN), jnp.float32),
    )(h, W_r, b2)


def _edge_sc_kernel(g_hbm, xr_hbm, src_hbm, dst_hbm, out_hbm,
                    src_v, dst_v, g_v, x_v, o_v, sg, sx, so):
    wid = lax.axis_index("s") * NC + lax.axis_index("c")
    base = wid * EW
    sidx, didx = src_v[0], dst_v[0]
    pltpu.sync_copy(src_hbm.at[pl.ds(base, EW)], sidx)
    pltpu.sync_copy(dst_hbm.at[pl.ds(base, EW)], didx)

    def gather_start(b, off):
        pltpu.make_async_copy(
            g_hbm.at[sidx.at[pl.ds(off, CH)]], g_v[b], sg[b]).start()
        pltpu.make_async_copy(
            xr_hbm.at[didx.at[pl.ds(off, CH)]], x_v[b], sx[b]).start()

    def gather_wait(b, off):
        pltpu.make_async_copy(
            g_hbm.at[sidx.at[pl.ds(off, CH)]], g_v[b], sg[b]).wait()
        pltpu.make_async_copy(
            xr_hbm.at[didx.at[pl.ds(off, CH)]], x_v[b], sx[b]).wait()

    def store_start(b, off):
        pltpu.make_async_copy(
            o_v[b], out_hbm.at[pl.ds(base + off, CH)], so[b]).start()

    def store_wait(b):
        pltpu.make_async_copy(
            o_v[b], out_hbm.at[pl.ds(base, CH)], so[b]).wait()

    def compute(b):
        gb, xb, ob = g_v[b], x_v[b], o_v[b]

        @plsc.parallel_loop(0, CH, unroll=8)
        def row_body(r):
            for j in range(NGRP):
                hh = gb[r, pl.ds(j * LANES, LANES)]
                uu = gb[r, pl.ds(HIDDEN + j * LANES, LANES)]
                xx = xb[r, pl.ds(j * LANES, LANES)]
                ob[r, pl.ds(j * LANES, LANES)] = hh / (1.0 + jnp.exp(-(xx + uu)))

    gather_start(0, 0)

    def pair_body(k, carry):
        i0 = 2 * k * CH
        gather_wait(0, i0)
        gather_start(1, i0 + CH)

        @pl.when(k > 0)
        def _():
            store_wait(0)
        compute(0)
        store_start(0, i0)

        gather_wait(1, i0 + CH)
        gather_start(0, i0 + 2 * CH)

        @pl.when(k > 0)
        def _():
            store_wait(1)
        compute(1)
        store_start(1, i0 + CH)
        return carry

    lax.fori_loop(0, (NCH - 1) // 2, pair_body, 0)

    last = (NCH - 1) * CH
    gather_wait(0, last)
    store_wait(0)
    compute(0)
    store_start(0, last)
    store_wait(1)
    store_wait(0)


@functools.partial(
    pl.kernel,
    out_type=jax.ShapeDtypeStruct((N_EDGES, HIDDEN), jnp.float32),
    mesh=plsc.VectorSubcoreMesh(core_axis_name="c", subcore_axis_name="s",
                                num_cores=NC, num_subcores=NS),
    scratch_types=[
        [pltpu.VMEM((EW,), jnp.int32)],
        [pltpu.VMEM((EW,), jnp.int32)],
        [pltpu.VMEM((CH, 2 * HIDDEN), jnp.float32) for _ in range(2)],
        [pltpu.VMEM((CH, HIDDEN), jnp.float32) for _ in range(2)],
        [pltpu.VMEM((CH, HIDDEN), jnp.float32) for _ in range(2)],
        [pltpu.SemaphoreType.DMA for _ in range(2)],
        [pltpu.SemaphoreType.DMA for _ in range(2)],
        [pltpu.SemaphoreType.DMA for _ in range(2)],
    ],
)
def _edge_messages(g_hbm, xr_hbm, src_hbm, dst_hbm, out_hbm,
                   src_v, dst_v, g_v, x_v, o_v, sg, sx, so):
    _edge_sc_kernel(g_hbm, xr_hbm, src_hbm, dst_hbm, out_hbm,
                    src_v, dst_v, g_v, x_v, o_v, sg, sx, so)


def kernel(h, src_x_r, edge_index, W_r, b_r):
    src = edge_index[0].astype(jnp.int32)
    dst = edge_index[1].astype(jnp.int32)
    g = _build_gates(h, W_r, b_r.reshape(1, HIDDEN))
    return _edge_messages(g, src_x_r, src, dst)


# speedup vs baseline: 1.1978x; 1.1978x over previous
import functools

import jax
import jax.numpy as jnp
from jax import lax
from jax.experimental import pallas as pl
from jax.experimental.pallas import tpu as pltpu
from jax.experimental.pallas import tpu_sc as plsc

HIDDEN = 128
N_NODES = 10000
N_EDGES = 320000

NC = 2
NS = 16
NW = NC * NS
EW = N_EDGES // NW
CH = 80
NCH = EW // CH
LANES = 16
NGRP = HIDDEN // LANES


def _gates_tc_kernel(h_ref, w_ref, b_ref, g_ref):
    hb = h_ref[...]
    u = lax.dot_general(hb, w_ref[...], (((1,), (1,)), ((), ())),
                        preferred_element_type=jnp.float32) + b_ref[...]
    g_ref[:, :HIDDEN] = hb
    g_ref[:, HIDDEN:] = u


def _build_gates(h, W_r, b2):
    rows = 1000
    return pl.pallas_call(
        _gates_tc_kernel,
        grid=(N_NODES // rows,),
        in_specs=[
            pl.BlockSpec((rows, HIDDEN), lambda i: (i, 0)),
            pl.BlockSpec((HIDDEN, HIDDEN), lambda i: (0, 0)),
            pl.BlockSpec((1, HIDDEN), lambda i: (0, 0)),
        ],
        out_specs=pl.BlockSpec((rows, 2 * HIDDEN), lambda i: (i, 0)),
        out_shape=jax.ShapeDtypeStruct((N_NODES, 2 * HIDDEN), jnp.float32),
    )(h, W_r, b2)


def _edge_sc_kernel(g_hbm, xr_hbm, src_hbm, dst_hbm, out_hbm,
                    src_v, dst_v, g_v, x_v, o_v, sg, sx, so):
    wid = lax.axis_index("s") * NC + lax.axis_index("c")
    base = wid * EW
    sidx, didx = src_v[0], dst_v[0]
    pltpu.sync_copy(src_hbm.at[pl.ds(base, EW)], sidx)
    pltpu.sync_copy(dst_hbm.at[pl.ds(base, EW)], didx)

    def gather_start(b, off):
        pltpu.make_async_copy(
            g_hbm.at[sidx.at[pl.ds(off, CH)]], g_v[b], sg[b]).start()
        pltpu.make_async_copy(
            xr_hbm.at[didx.at[pl.ds(off, CH)]], x_v[b], sx[b]).start()

    def gather_wait(b, off):
        pltpu.make_async_copy(
            g_hbm.at[sidx.at[pl.ds(off, CH)]], g_v[b], sg[b]).wait()
        pltpu.make_async_copy(
            xr_hbm.at[didx.at[pl.ds(off, CH)]], x_v[b], sx[b]).wait()

    def store_start(b, off):
        pltpu.make_async_copy(
            o_v[b], out_hbm.at[pl.ds(base + off, CH)], so[b]).start()

    def store_wait(b):
        pltpu.make_async_copy(
            o_v[b], out_hbm.at[pl.ds(base, CH)], so[b]).wait()

    def compute(b):
        gb, xb, ob = g_v[b], x_v[b], o_v[b]

        @plsc.parallel_loop(0, CH, unroll=2)
        def row_body(r):
            for j in range(NGRP):
                hh = gb[r, pl.ds(j * LANES, LANES)]
                uu = gb[r, pl.ds(HIDDEN + j * LANES, LANES)]
                xx = xb[r, pl.ds(j * LANES, LANES)]
                ob[r, pl.ds(j * LANES, LANES)] = hh / (1.0 + jnp.exp(-(xx + uu)))

    gather_start(0, 0)

    def pair_body(k, carry):
        i0 = 2 * k * CH
        gather_wait(0, i0)
        gather_start(1, i0 + CH)

        @pl.when(k > 0)
        def _():
            store_wait(0)
        compute(0)
        store_start(0, i0)

        gather_wait(1, i0 + CH)
        gather_start(0, i0 + 2 * CH)

        @pl.when(k > 0)
        def _():
            store_wait(1)
        compute(1)
        store_start(1, i0 + CH)
        return carry

    lax.fori_loop(0, (NCH - 1) // 2, pair_body, 0)

    last = (NCH - 1) * CH
    gather_wait(0, last)
    store_wait(0)
    compute(0)
    store_start(0, last)
    store_wait(1)
    store_wait(0)


@functools.partial(
    pl.kernel,
    out_type=jax.ShapeDtypeStruct((N_EDGES, HIDDEN), jnp.float32),
    mesh=plsc.VectorSubcoreMesh(core_axis_name="c", subcore_axis_name="s",
                                num_cores=NC, num_subcores=NS),
    scratch_types=[
        [pltpu.VMEM((EW,), jnp.int32)],
        [pltpu.VMEM((EW,), jnp.int32)],
        [pltpu.VMEM((CH, 2 * HIDDEN), jnp.float32) for _ in range(2)],
        [pltpu.VMEM((CH, HIDDEN), jnp.float32) for _ in range(2)],
        [pltpu.VMEM((CH, HIDDEN), jnp.float32) for _ in range(2)],
        [pltpu.SemaphoreType.DMA for _ in range(2)],
        [pltpu.SemaphoreType.DMA for _ in range(2)],
        [pltpu.SemaphoreType.DMA for _ in range(2)],
    ],
)
def _edge_messages(g_hbm, xr_hbm, src_hbm, dst_hbm, out_hbm,
                   src_v, dst_v, g_v, x_v, o_v, sg, sx, so):
    _edge_sc_kernel(g_hbm, xr_hbm, src_hbm, dst_hbm, out_hbm,
                    src_v, dst_v, g_v, x_v, o_v, sg, sx, so)


def kernel(h, src_x_r, edge_index, W_r, b_r):
    src = edge_index[0].astype(jnp.int32)
    dst = edge_index[1].astype(jnp.int32)
    g = _build_gates(h, W_r, b_r.reshape(1, HIDDEN))
    return _edge_messages(g, src_x_r, src, dst)


# speedup vs baseline: 1.2029x; 1.0043x over previous
import functools

import jax
import jax.numpy as jnp
from jax import lax
from jax.experimental import pallas as pl
from jax.experimental.pallas import tpu as pltpu
from jax.experimental.pallas import tpu_sc as plsc

HIDDEN = 128
N_NODES = 10000
N_EDGES = 320000

NC = 2
NS = 16
NW = NC * NS
EW = N_EDGES // NW
CH = 80
NCH = EW // CH
LANES = 16
NGRP = HIDDEN // LANES


def _gates_tc_kernel(h_ref, w_ref, b_ref, g_ref):
    hb = h_ref[...]
    u = lax.dot_general(hb, w_ref[...], (((1,), (1,)), ((), ())),
                        preferred_element_type=jnp.float32) + b_ref[...]
    g_ref[:, :HIDDEN] = hb
    g_ref[:, HIDDEN:] = u


def _build_gates(h, W_r, b2):
    rows = 1000
    return pl.pallas_call(
        _gates_tc_kernel,
        grid=(N_NODES // rows,),
        in_specs=[
            pl.BlockSpec((rows, HIDDEN), lambda i: (i, 0)),
            pl.BlockSpec((HIDDEN, HIDDEN), lambda i: (0, 0)),
            pl.BlockSpec((1, HIDDEN), lambda i: (0, 0)),
        ],
        out_specs=pl.BlockSpec((rows, 2 * HIDDEN), lambda i: (i, 0)),
        out_shape=jax.ShapeDtypeStruct((N_NODES, 2 * HIDDEN), jnp.float32),
    )(h, W_r, b2)


def _edge_sc_kernel(g_hbm, xr_hbm, src_hbm, dst_hbm, out_hbm,
                    src_v, dst_v, g_v, x_v, o_v, sg, sx, so):
    wid = lax.axis_index("s") * NC + lax.axis_index("c")
    base = wid * EW
    sidx, didx = src_v[0], dst_v[0]
    pltpu.sync_copy(src_hbm.at[pl.ds(base, EW)], sidx)
    pltpu.sync_copy(dst_hbm.at[pl.ds(base, EW)], didx)

    def gather_start(b, off):
        pltpu.make_async_copy(
            g_hbm.at[sidx.at[pl.ds(off, CH)]], g_v[b], sg[b]).start()
        pltpu.make_async_copy(
            xr_hbm.at[didx.at[pl.ds(off, CH)]], x_v[b], sx[b]).start()

    def gather_wait(b, off):
        pltpu.make_async_copy(
            g_hbm.at[sidx.at[pl.ds(off, CH)]], g_v[b], sg[b]).wait()
        pltpu.make_async_copy(
            xr_hbm.at[didx.at[pl.ds(off, CH)]], x_v[b], sx[b]).wait()

    def store_start(b, off):
        pltpu.make_async_copy(
            o_v[b], out_hbm.at[pl.ds(base + off, CH)], so[b]).start()

    def store_wait(b):
        pltpu.make_async_copy(
            o_v[b], out_hbm.at[pl.ds(base, CH)], so[b]).wait()

    def compute(b):
        gb, xb, ob = g_v[b], x_v[b], o_v[b]

        @plsc.parallel_loop(0, CH, unroll=1)
        def row_body(r):
            for j in range(NGRP):
                hh = gb[r, pl.ds(j * LANES, LANES)]
                uu = gb[r, pl.ds(HIDDEN + j * LANES, LANES)]
                xx = xb[r, pl.ds(j * LANES, LANES)]
                ob[r, pl.ds(j * LANES, LANES)] = hh / (1.0 + jnp.exp(-(xx + uu)))

    gather_start(0, 0)

    def pair_body(k, carry):
        i0 = 2 * k * CH
        gather_wait(0, i0)
        gather_start(1, i0 + CH)

        @pl.when(k > 0)
        def _():
            store_wait(0)
        compute(0)
        store_start(0, i0)

        gather_wait(1, i0 + CH)
        gather_start(0, i0 + 2 * CH)

        @pl.when(k > 0)
        def _():
            store_wait(1)
        compute(1)
        store_start(1, i0 + CH)
        return carry

    lax.fori_loop(0, (NCH - 1) // 2, pair_body, 0)

    last = (NCH - 1) * CH
    gather_wait(0, last)
    store_wait(0)
    compute(0)
    store_start(0, last)
    store_wait(1)
    store_wait(0)


@functools.partial(
    pl.kernel,
    out_type=jax.ShapeDtypeStruct((N_EDGES, HIDDEN), jnp.float32),
    mesh=plsc.VectorSubcoreMesh(core_axis_name="c", subcore_axis_name="s",
                                num_cores=NC, num_subcores=NS),
    scratch_types=[
        [pltpu.VMEM((EW,), jnp.int32)],
        [pltpu.VMEM((EW,), jnp.int32)],
        [pltpu.VMEM((CH, 2 * HIDDEN), jnp.float32) for _ in range(2)],
        [pltpu.VMEM((CH, HIDDEN), jnp.float32) for _ in range(2)],
        [pltpu.VMEM((CH, HIDDEN), jnp.float32) for _ in range(2)],
        [pltpu.SemaphoreType.DMA for _ in range(2)],
        [pltpu.SemaphoreType.DMA for _ in range(2)],
        [pltpu.SemaphoreType.DMA for _ in range(2)],
    ],
)
def _edge_messages(g_hbm, xr_hbm, src_hbm, dst_hbm, out_hbm,
                   src_v, dst_v, g_v, x_v, o_v, sg, sx, so):
    _edge_sc_kernel(g_hbm, xr_hbm, src_hbm, dst_hbm, out_hbm,
                    src_v, dst_v, g_v, x_v, o_v, sg, sx, so)


def kernel(h, src_x_r, edge_index, W_r, b_r):
    src = edge_index[0].astype(jnp.int32)
    dst = edge_index[1].astype(jnp.int32)
    g = _build_gates(h, W_r, b_r.reshape(1, HIDDEN))
    return _edge_messages(g, src_x_r, src, dst)


# speedup vs baseline: 1.4250x; 1.1846x over previous
import functools

import jax
import jax.numpy as jnp
from jax import lax
from jax.experimental import pallas as pl
from jax.experimental.pallas import tpu as pltpu
from jax.experimental.pallas import tpu_sc as plsc

HIDDEN = 128
N_NODES = 10000
N_EDGES = 320000

NC = 2
NS = 16
NW = NC * NS
EW = N_EDGES // NW
CH = 80
NCH = EW // CH
LANES = 16
GW = HIDDEN
XW = HIDDEN


def _gates_tc_kernel(h_ref, w_ref, b_ref, u_ref):
    u_ref[...] = lax.dot_general(
        h_ref[...], w_ref[...], (((1,), (1,)), ((), ())),
        preferred_element_type=jnp.float32) + b_ref[...]


def _build_gates(h, W_r, b2):
    rows = 1000
    return pl.pallas_call(
        _gates_tc_kernel,
        grid=(N_NODES // rows,),
        in_specs=[
            pl.BlockSpec((rows, HIDDEN), lambda i: (i, 0)),
            pl.BlockSpec((HIDDEN, HIDDEN), lambda i: (0, 0)),
            pl.BlockSpec((1, HIDDEN), lambda i: (0, 0)),
        ],
        out_specs=pl.BlockSpec((rows, HIDDEN), lambda i: (i, 0)),
        out_shape=jax.ShapeDtypeStruct((N_NODES, HIDDEN), jnp.float32),
    )(h, W_r, b2)


def _pack_ileaved(a):
    n, c = a.shape
    ab = a.astype(jnp.bfloat16).reshape(n, c // 32, 2, LANES).transpose(0, 1, 3, 2)
    return lax.bitcast_convert_type(ab, jnp.int32).reshape(n, c // 2)


def _edge_sc_kernel(g_hbm, xr_hbm, src_hbm, dst_hbm, out_hbm,
                    src_v, dst_v, g_v, x_v, o_v, sg, sx, so):
    wid = lax.axis_index("s") * NC + lax.axis_index("c")
    base = wid * EW
    sidx, didx = src_v[0], dst_v[0]
    pltpu.sync_copy(src_hbm.at[pl.ds(base, EW)], sidx)
    pltpu.sync_copy(dst_hbm.at[pl.ds(base, EW)], didx)

    def gather_start(b, off):
        pltpu.make_async_copy(
            g_hbm.at[sidx.at[pl.ds(off, CH)]], g_v[b], sg[b]).start()
        pltpu.make_async_copy(
            xr_hbm.at[didx.at[pl.ds(off, CH)]], x_v[b], sx[b]).start()

    def gather_wait(b, off):
        pltpu.make_async_copy(
            g_hbm.at[sidx.at[pl.ds(off, CH)]], g_v[b], sg[b]).wait()
        pltpu.make_async_copy(
            xr_hbm.at[didx.at[pl.ds(off, CH)]], x_v[b], sx[b]).wait()

    def store_start(b, off):
        pltpu.make_async_copy(
            o_v[b], out_hbm.at[pl.ds(base + off, CH)], so[b]).start()

    def store_wait(b):
        pltpu.make_async_copy(
            o_v[b], out_hbm.at[pl.ds(base, CH)], so[b]).wait()

    def _unpk(w):
        lo = lax.bitcast_convert_type(w << 16, jnp.float32)
        hi = lax.bitcast_convert_type(w & jnp.int32(-65536), jnp.float32)
        return lo, hi

    def compute(b):
        gb, xb, ob = g_v[b], x_v[b], o_v[b]

        @plsc.parallel_loop(0, CH, unroll=1)
        def row_body(r):
            for j in range(HIDDEN // 32):
                h0, h1 = _unpk(gb[r, pl.ds(j * LANES, LANES)])
                u0, u1 = _unpk(gb[r, pl.ds(GW // 2 + j * LANES, LANES)])
                x0 = xb[r, pl.ds(j * 2 * LANES, LANES)]
                x1 = xb[r, pl.ds(j * 2 * LANES + LANES, LANES)]
                ob[r, pl.ds(j * 2 * LANES, LANES)] = (
                    h0 / (1.0 + jnp.exp(-(x0 + u0))))
                ob[r, pl.ds(j * 2 * LANES + LANES, LANES)] = (
                    h1 / (1.0 + jnp.exp(-(x1 + u1))))

    gather_start(0, 0)

    def pair_body(k, carry):
        i0 = 2 * k * CH
        gather_wait(0, i0)
        gather_start(1, i0 + CH)

        @pl.when(k > 0)
        def _():
            store_wait(0)
        compute(0)
        store_start(0, i0)

        gather_wait(1, i0 + CH)
        gather_start(0, i0 + 2 * CH)

        @pl.when(k > 0)
        def _():
            store_wait(1)
        compute(1)
        store_start(1, i0 + CH)
        return carry

    lax.fori_loop(0, (NCH - 1) // 2, pair_body, 0)

    last = (NCH - 1) * CH
    gather_wait(0, last)
    store_wait(0)
    compute(0)
    store_start(0, last)
    store_wait(1)
    store_wait(0)


@functools.partial(
    pl.kernel,
    out_type=jax.ShapeDtypeStruct((N_EDGES, HIDDEN), jnp.float32),
    mesh=plsc.VectorSubcoreMesh(core_axis_name="c", subcore_axis_name="s",
                                num_cores=NC, num_subcores=NS),
    scratch_types=[
        [pltpu.VMEM((EW,), jnp.int32)],
        [pltpu.VMEM((EW,), jnp.int32)],
        [pltpu.VMEM((CH, GW), jnp.int32) for _ in range(2)],
        [pltpu.VMEM((CH, XW), jnp.float32) for _ in range(2)],
        [pltpu.VMEM((CH, HIDDEN), jnp.float32) for _ in range(2)],
        [pltpu.SemaphoreType.DMA for _ in range(2)],
        [pltpu.SemaphoreType.DMA for _ in range(2)],
        [pltpu.SemaphoreType.DMA for _ in range(2)],
    ],
)
def _edge_messages(g_hbm, xr_hbm, src_hbm, dst_hbm, out_hbm,
                   src_v, dst_v, g_v, x_v, o_v, sg, sx, so):
    _edge_sc_kernel(g_hbm, xr_hbm, src_hbm, dst_hbm, out_hbm,
                    src_v, dst_v, g_v, x_v, o_v, sg, sx, so)


def kernel(h, src_x_r, edge_index, W_r, b_r):
    src = edge_index[0].astype(jnp.int32)
    dst = edge_index[1].astype(jnp.int32)
    u = _build_gates(h, W_r, b_r.reshape(1, HIDDEN))
    g_words = _pack_ileaved(jnp.concatenate([h, u], axis=1))
    return _edge_messages(g_words, src_x_r, src, dst)
